# Initial kernel scaffold; baseline (speedup 1.0000x reference)
import jax
import jax.numpy as jnp
from jax.experimental import pallas as pl


def kernel(component_mole_frac, component_batch_batch, ln_gamma_calc):
    raise NotImplementedError("write your pallas kernel here")


# speedup vs baseline: 147.0796x; 147.0796x over previous
import jax
import jax.numpy as jnp
from jax import lax
from jax.experimental import pallas as pl
from jax.experimental.pallas import tpu as pltpu
from jax.experimental.pallas import tpu_sc as plsc

N = 2_000_000
B = 500_000
RT = 8.31446261815324 * 298.15

NC, NS, L = 2, 16, 16
ROWS = N // 128
SPLIT = 7813
BPAD = 512_000
BS = BPAD // NS
ZCH = 8000
PCH = 4000
G = 32
FULL_I = 15


def _sc_body(ids2_hbm, lg_hbm, out_hbm, idx_v, lg_v, vj_v, ones_v, zbuf,
             bst_v, rec_v, obuf, ssq_v, sum_acc, cnt_acc):
    c = lax.axis_index("c")
    s = lax.axis_index("s")
    iota = lax.broadcasted_iota(jnp.int32, (L,), 0)

    def _zfill(i, _):
        zbuf[pl.ds(i * L, L)] = jnp.zeros((L,), jnp.float32)
        return 0
    lax.fori_loop(0, ZCH // L, _zfill, 0)
    for k in range(128 // L):
        ones_v[pl.ds(k * L, L)] = jnp.ones((L,), jnp.float32)
    zbase = s * BS
    for p in range(BS // ZCH):
        pltpu.sync_copy(zbuf, sum_acc.at[pl.ds(zbase + p * ZCH, ZCH)])
        pltpu.sync_copy(zbuf, cnt_acc.at[pl.ds(zbase + p * ZCH, ZCH)])
    plsc.subcore_barrier()

    base = jnp.where(c == 0, 0, SPLIT)
    rem = jnp.where(c == 0, 5, 4)
    start = base + 488 * s + jnp.minimum(s, rem)
    tail9 = s < rem

    ssq_v[...] = jnp.zeros((L,), jnp.float32)

    def _do_row(j):
        acc = ssq_v[...]
        for k in range(128 // L):
            lgv = lg_v[pl.ds(j * 128 + k * L, L)]
            g = lgv * RT
            cot = jnp.ones((L,), jnp.float32)
            vj = cot * g - g
            vj_v[pl.ds(j * 128 + k * L, L)] = vj
            acc = acc + vj * vj
        ssq_v[...] = acc
        pltpu.sync_copy(vj_v.at[pl.ds(j * 128, 128)],
                        sum_acc.at[idx_v.at[j]], add=True)
        pltpu.sync_copy(ones_v, cnt_acc.at[idx_v.at[j]], add=True)

    def _row_loop(j, carry):
        _do_row(j)
        return carry

    for i in range(FULL_I):
        row = start + G * i
        pltpu.sync_copy(ids2_hbm.at[pl.ds(row, G)], idx_v)
        pltpu.sync_copy(lg_hbm.at[pl.ds(row * 128, G * 128)], lg_v)
        lax.fori_loop(0, G, _row_loop, 0)
    trow = start + G * FULL_I
    pltpu.sync_copy(ids2_hbm.at[pl.ds(trow, 8)], idx_v.at[pl.ds(0, 8)])
    pltpu.sync_copy(lg_hbm.at[pl.ds(trow * 128, 8 * 128)],
                    lg_v.at[pl.ds(0, 8 * 128)])
    lax.fori_loop(0, 8, _row_loop, 0)

    @pl.when(tail9)
    def _tail():
        pltpu.sync_copy(ids2_hbm.at[pl.ds(trow + 8, 1)],
                        idx_v.at[pl.ds(8, 1)])
        pltpu.sync_copy(lg_hbm.at[pl.ds((trow + 8) * 128, 128)],
                        lg_v.at[pl.ds(8 * 128, 128)])
        _do_row(8)

    plsc.subcore_barrier()

    pltpu.sync_copy(ids2_hbm.at[pl.ds(SPLIT, 1)], bst_v)
    bstar = bst_v[0, pl.ds(0, L)][0]

    def _p2_step(q, carry):
        t2, off = carry
        sv = lg_v[pl.ds(q * L, L)]
        cv = vj_v[pl.ds(q * L, L)]
        gidx = off + q * L + iota
        mask = jnp.where(c == 0, gidx < bstar, gidx > bstar)
        contrib = jnp.where(mask, sv * sv / jnp.maximum(cv, 1.0),
                            jnp.zeros((L,), jnp.float32))
        return (t2 + contrib, off)

    t2 = jnp.zeros((L,), jnp.float32)
    for p in range(BS // PCH):
        off = s * BS + p * PCH
        pltpu.sync_copy(sum_acc.at[pl.ds(off, PCH)], lg_v.at[pl.ds(0, PCH)])
        pltpu.sync_copy(cnt_acc.at[pl.ds(off, PCH)], vj_v.at[pl.ds(0, PCH)])
        t2, _ = lax.fori_loop(0, PCH // L, _p2_step, (t2, off))

    b8 = (bstar // 8) * 8
    lane = bstar - b8
    pltpu.sync_copy(sum_acc.at[pl.ds(b8, L)], rec_v.at[pl.ds(0, L)])
    pltpu.sync_copy(cnt_acc.at[pl.ds(b8, L)], rec_v.at[pl.ds(L, L)])
    srec_v = rec_v[pl.ds(0, L)]
    crec_v = rec_v[pl.ds(L, L)]
    lmask = iota == lane
    zero = jnp.zeros((L,), jnp.float32)
    s_rec = jnp.sum(jnp.where(lmask, srec_v, zero))
    c_rec = jnp.sum(jnp.where(lmask, crec_v, zero))
    is_s0 = (s == 0).astype(jnp.float32)

    t2_tot = jnp.sum(t2)
    ssq_tot = jnp.sum(ssq_v[...])
    res = (jnp.where(iota == 0, t2_tot, zero)
           + jnp.where(iota == 1, ssq_tot, zero)
           + jnp.where(iota == 2, s_rec * is_s0, zero)
           + jnp.where(iota == 3, c_rec * is_s0, zero))
    obuf[...] = res
    wid = c * NS + s
    pltpu.sync_copy(obuf, out_hbm.at[wid])


@jax.jit
def _gd_loss_sc(ids2, lg1):
    mesh = plsc.VectorSubcoreMesh(core_axis_name="c", subcore_axis_name="s")
    f = pl.kernel(
        _sc_body,
        out_type=jax.ShapeDtypeStruct((NC * NS, L), jnp.float32),
        mesh=mesh,
        compiler_params=pltpu.CompilerParams(use_tc_tiling_on_sc=False,
                                             needs_layout_passes=False),
        scratch_types=[
            pltpu.VMEM((G, 128), jnp.int32),
            pltpu.VMEM((G * 128,), jnp.float32),
            pltpu.VMEM((G * 128,), jnp.float32),
            pltpu.VMEM((128,), jnp.float32),
            pltpu.VMEM((ZCH,), jnp.float32),
            pltpu.VMEM((1, 128), jnp.int32),
            pltpu.VMEM((2 * L,), jnp.float32),
            pltpu.VMEM((L,), jnp.float32),
            pltpu.VMEM((L,), jnp.float32),
            pltpu.VMEM_SHARED((BPAD,), jnp.float32),
            pltpu.VMEM_SHARED((BPAD,), jnp.float32),
        ],
    )
    rows = f(ids2, lg1)
    t2 = jnp.sum(rows[:, 0])
    ssq = jnp.sum(rows[:, 1])
    s_b = jnp.sum(rows[:, 2])
    c_b = jnp.sum(rows[:, 3])
    t2 = t2 + s_b * s_b / jnp.maximum(c_b, 1.0)
    return (ssq - t2) / jnp.float32(B)


def kernel(component_mole_frac, component_batch_batch, ln_gamma_calc):
    del component_mole_frac
    ids2 = component_batch_batch.reshape(ROWS, 128)
    lg1 = ln_gamma_calc.reshape(N)
    return _gd_loss_sc(ids2, lg1)
